# baseline (speedup 1.0000x reference)
import functools

import jax
import jax.numpy as jnp
from jax import lax
from jax.experimental import pallas as pl
from jax.experimental.pallas import tpu as pltpu
from jax.experimental.pallas import tpu_sc as plsc

VOCAB = 1000000
EMB = 64
HID = 1024
OUT = 128
NTOK = 16384

NW = 32
BPW = NTOK // NW
CHUNK = 128
NCH = BPW // CHUNK
LANES = 16
NVEC = EMB // LANES


def _sc_partial_sums(idx, table):
    mesh = plsc.VectorSubcoreMesh(core_axis_name="c", subcore_axis_name="s")

    @functools.partial(
        pl.kernel,
        mesh=mesh,
        out_type=jax.ShapeDtypeStruct((NW, EMB), jnp.float32),
        compiler_params=pltpu.CompilerParams(use_tc_tiling_on_sc=False),
        scratch_types=[
            pltpu.VMEM((BPW,), jnp.int32),
            pltpu.VMEM((BPW, EMB), jnp.float32),
            pltpu.VMEM((1, EMB), jnp.float32),
            pltpu.SemaphoreType.DMA,
        ],
    )
    def k(idx_hbm, table_hbm, out_hbm, idx_v, rows_v, acc_v, sem):
        wid = lax.axis_index("s") * 2 + lax.axis_index("c")
        base = wid * BPW
        pltpu.sync_copy(idx_hbm.at[pl.ds(base, BPW)], idx_v)
        copies = []
        for j in range(NCH):
            copies.append(
                pltpu.async_copy(
                    table_hbm.at[idx_v.at[pl.ds(j * CHUNK, CHUNK)]],
                    rows_v.at[pl.ds(j * CHUNK, CHUNK)],
                    sem,
                )
            )
        for c in copies:
            c.wait()

        def body(r, carry):
            return tuple(
                carry[c] + rows_v[r, pl.ds(c * LANES, LANES)]
                for c in range(NVEC)
            )

        zero = jnp.zeros((LANES,), jnp.float32)
        acc = lax.fori_loop(0, BPW, body, (zero,) * NVEC)
        for c in range(NVEC):
            acc_v[0, pl.ds(c * LANES, LANES)] = acc[c]
        pltpu.sync_copy(acc_v, out_hbm.at[pl.ds(wid, 1)])

    return k(idx, table)


def _tc_ffnn(partials, W1, b1, W2, b2):
    def body(p_ref, w1_ref, b1_ref, w2_ref, b2_ref, o_ref):
        bag = jnp.sum(p_ref[...], axis=0, keepdims=True) * (1.0 / NTOK)
        h = jnp.dot(bag, w1_ref[...], preferred_element_type=jnp.float32)
        h = jnp.maximum(h + b1_ref[...], 0.0)
        logits = jnp.dot(h, w2_ref[...], preferred_element_type=jnp.float32)
        logits = logits + b2_ref[...]
        m = jnp.max(logits, axis=-1, keepdims=True)
        lse = jnp.log(jnp.sum(jnp.exp(logits - m), axis=-1, keepdims=True)) + m
        o_ref[...] = logits - lse

    return pl.pallas_call(
        body,
        out_shape=jax.ShapeDtypeStruct((1, OUT), jnp.float32),
    )(partials, W1, b1.reshape(1, HID), W2, b2.reshape(1, OUT))


def kernel(input, embeddings, W1, b1, W2, b2):
    partials = _sc_partial_sums(input, embeddings)
    return _tc_ffnn(partials, W1, b1, W2, b2)


# speedup vs baseline: 1.6975x; 1.6975x over previous
import functools

import jax
import jax.numpy as jnp
from jax import lax
from jax.experimental import pallas as pl
from jax.experimental.pallas import tpu as pltpu
from jax.experimental.pallas import tpu_sc as plsc

VOCAB = 1000000
EMB = 64
HID = 1024
OUT = 128
NTOK = 16384

NW = 32
BPW = NTOK // NW
CHUNK = 32
NCH = BPW // CHUNK
LANES = 16
NVEC = EMB // LANES


def _sc_partial_sums(idx, table):
    mesh = plsc.VectorSubcoreMesh(core_axis_name="c", subcore_axis_name="s")

    @functools.partial(
        pl.kernel,
        mesh=mesh,
        out_type=jax.ShapeDtypeStruct((NW, EMB), jnp.float32),
        scratch_types=[
            pltpu.VMEM((BPW,), jnp.int32),
            pltpu.VMEM((2, CHUNK, EMB), jnp.float32),
            pltpu.VMEM((1, EMB), jnp.float32),
            pltpu.SemaphoreType.DMA,
            pltpu.SemaphoreType.DMA,
        ],
    )
    def k(idx_hbm, table_hbm, out_hbm, idx_v, buf_v, acc_v, sem0, sem1):
        wid = lax.axis_index("s") * 2 + lax.axis_index("c")
        base = wid * BPW
        pltpu.sync_copy(idx_hbm.at[pl.ds(base, BPW)], idx_v)
        sems = (sem0, sem1)

        def fire(c, slot, sem):
            for g in range(CHUNK // LANES):
                v = idx_v[pl.ds(c * CHUNK + g * LANES, LANES)]
                for l in range(LANES):
                    pltpu.async_copy(
                        table_hbm.at[pl.ds(v[l], 1)],
                        buf_v.at[slot, pl.ds(g * LANES + l, 1)],
                        sem,
                    )

        def drain(slot, sem):
            pltpu.make_async_copy(
                table_hbm.at[pl.ds(0, CHUNK)], buf_v.at[slot], sem
            ).wait()

        def accum(slot, acc):
            def ab(j, carry):
                return tuple(
                    carry[v] + buf_v[slot, j, pl.ds(v * LANES, LANES)]
                    for v in range(NVEC)
                )
            return lax.fori_loop(0, CHUNK, ab, acc)

        zero = jnp.zeros((LANES,), jnp.float32)
        acc = (zero,) * NVEC
        fire(0, 0, sems[0])
        for c in range(NCH):
            slot = c % 2
            if c + 1 < NCH:
                fire(c + 1, (c + 1) % 2, sems[(c + 1) % 2])
            drain(slot, sems[slot])
            acc = accum(slot, acc)
        for v in range(NVEC):
            acc_v[0, pl.ds(v * LANES, LANES)] = acc[v]
        pltpu.sync_copy(acc_v, out_hbm.at[pl.ds(wid, 1)])

    return k(idx, table)


def _tc_ffnn(partials, W1, b1, W2, b2):
    def body(p_ref, w1_ref, b1_ref, w2_ref, b2_ref, o_ref):
        bag = jnp.sum(p_ref[...], axis=0, keepdims=True) * (1.0 / NTOK)
        h = jnp.dot(bag, w1_ref[...], preferred_element_type=jnp.float32)
        h = jnp.maximum(h + b1_ref[...], 0.0)
        logits = jnp.dot(h, w2_ref[...], preferred_element_type=jnp.float32)
        logits = logits + b2_ref[...]
        m = jnp.max(logits, axis=-1, keepdims=True)
        lse = jnp.log(jnp.sum(jnp.exp(logits - m), axis=-1, keepdims=True)) + m
        o_ref[...] = logits - lse

    return pl.pallas_call(
        body,
        out_shape=jax.ShapeDtypeStruct((1, OUT), jnp.float32),
    )(partials, W1, b1.reshape(1, HID), W2, b2.reshape(1, OUT))


def kernel(input, embeddings, W1, b1, W2, b2):
    partials = _sc_partial_sums(input, embeddings)
    return _tc_ffnn(partials, W1, b1, W2, b2)
